# Initial kernel scaffold; baseline (speedup 1.0000x reference)
import jax
import jax.numpy as jnp
from jax.experimental import pallas as pl


def kernel(x, x_features, src_mask, mel_mask, duration_target, pitch_target, energy_target, max_len, dur_params, pitch_params, energy_params, pitch_bins, energy_bins, pitch_embedding, energy_embedding):
    raise NotImplementedError("write your pallas kernel here")


# speedup vs baseline: 35.4120x; 35.4120x over previous
import jax
import jax.numpy as jnp
from jax.experimental import pallas as pl
from jax.experimental.pallas import tpu as pltpu

_B, _S, _ML, _D, _F = 16, 512, 2048, 256, 256


def _conv3(xin, w_ref, b_ref):
    zero = jnp.zeros((1, xin.shape[1]), jnp.float32)
    xprev = jnp.concatenate([zero, xin[:-1]], axis=0)
    xnext = jnp.concatenate([xin[1:], zero], axis=0)
    y = jnp.dot(xprev, w_ref[0], preferred_element_type=jnp.float32)
    y = y + jnp.dot(xin, w_ref[1], preferred_element_type=jnp.float32)
    y = y + jnp.dot(xnext, w_ref[2], preferred_element_type=jnp.float32)
    return y + b_ref[:, :]


def _ln(h, g_ref, be_ref):
    m = jnp.mean(h, axis=-1, keepdims=True)
    d = h - m
    v = jnp.mean(d * d, axis=-1, keepdims=True)
    return d * jax.lax.rsqrt(v + 1e-5) * g_ref[:, :] + be_ref[:, :]


def _predictor(xin, w1, b1, g1, be1, w2, b2, g2, be2, lwt, lb):
    h = jnp.maximum(_conv3(xin, w1, b1), 0.0)
    h = _ln(h, g1, be1)
    h = jnp.maximum(_conv3(h, w2, b2), 0.0)
    h = _ln(h, g2, be2)
    out = jnp.sum(h * lwt[:, :], axis=-1, keepdims=True)
    return out + lb[:, :]


def _body(
    x_ref, xf_ref, dur_ref, pt_ref, et_ref,
    pbins_ref, ebins_ref, pemb_ref, eemb_ref,
    dw1, db1, dg1, dbe1, dw2, db2, dg2, dbe2, dlwt, dlb,
    pw1, pb1, pg1, pbe1, pw2, pb2, pg2, pbe2, plwt, plb,
    ew1, eb1, eg1, ebe1, ew2, eb2, eg2, ebe2, elwt, elb,
    xout_ref, logdur_ref, ppred_ref, epred_ref, mel_ref,
):
    x_b = x_ref[0]
    xf_b = xf_ref[0]
    dur_row = dur_ref[0].astype(jnp.float32)

    jj = jax.lax.broadcasted_iota(jnp.int32, (_S, _S), 0)
    ss = jax.lax.broadcasted_iota(jnp.int32, (_S, _S), 1)
    tri = (jj <= ss).astype(jnp.float32)
    cum_row = jnp.dot(dur_row, tri, preferred_element_type=jnp.float32)
    cumprev_row = cum_row - dur_row

    t_col = jax.lax.broadcasted_iota(jnp.int32, (_ML, 1), 0).astype(jnp.float32)
    g_lt_hi = (t_col < cum_row).astype(jnp.float32)
    g_lt_lo = (t_col < cumprev_row).astype(jnp.float32)
    gmat = g_lt_hi - g_lt_lo
    x_exp = jnp.dot(gmat, x_b, preferred_element_type=jnp.float32)
    xf_exp = jnp.dot(gmat, xf_b, preferred_element_type=jnp.float32)

    logdur_ref[0] = _predictor(xf_b, dw1, db1, dg1, dbe1, dw2, db2, dg2, dbe2, dlwt, dlb)

    ppred_ref[0] = _predictor(xf_exp, pw1, pb1, pg1, pbe1, pw2, pb2, pg2, pbe2, plwt, plb)
    epred_ref[0] = _predictor(xf_exp, ew1, eb1, eg1, ebe1, ew2, eb2, eg2, ebe2, elwt, elb)

    p_col = pt_ref[0]
    e_col = et_ref[0]
    pidx = jnp.sum((pbins_ref[:, :] < p_col).astype(jnp.float32), axis=-1, keepdims=True)
    eidx = jnp.sum((ebins_ref[:, :] < e_col).astype(jnp.float32), axis=-1, keepdims=True)
    j_row = jax.lax.broadcasted_iota(jnp.int32, (_ML, 256), 1).astype(jnp.float32)
    p_onehot = (pidx == j_row).astype(jnp.float32)
    e_onehot = (eidx == j_row).astype(jnp.float32)
    emb = jnp.dot(p_onehot, pemb_ref[:, :], preferred_element_type=jnp.float32)
    emb = emb + jnp.dot(e_onehot, eemb_ref[:, :], preferred_element_type=jnp.float32)

    xout_ref[0] = x_exp + emb
    mel_ref[0] = cum_row[:, _S - 128:]


def _pack_params(p):
    return [
        p['w1'], p['b1'].reshape(1, _F), p['g1'].reshape(1, _F), p['be1'].reshape(1, _F),
        p['w2'], p['b2'].reshape(1, _F), p['g2'].reshape(1, _F), p['be2'].reshape(1, _F),
        p['lw'].reshape(1, _F), p['lb'].reshape(1, 1),
    ]


def kernel(x, x_features, src_mask, mel_mask, duration_target, pitch_target,
           energy_target, max_len, dur_params, pitch_params, energy_params,
           pitch_bins, energy_bins, pitch_embedding, energy_embedding):
    B, S, D = x.shape
    ML = mel_mask.shape[1]

    dur3 = duration_target.reshape(B, 1, S)
    pt3 = pitch_target.reshape(B, ML, 1)
    et3 = energy_target.reshape(B, ML, 1)
    pad = jnp.full((1,), jnp.inf, jnp.float32)
    pbins = jnp.concatenate([pitch_bins, pad]).reshape(1, 256)
    ebins = jnp.concatenate([energy_bins, pad]).reshape(1, 256)

    batch3 = lambda i: (i, 0, 0)

    def full_spec(arr):
        return pl.BlockSpec(arr.shape, lambda i: (0,) * arr.ndim)

    in_specs = [
        pl.BlockSpec((1, S, D), batch3),
        pl.BlockSpec((1, S, D), batch3),
        pl.BlockSpec((1, 1, S), batch3),
        pl.BlockSpec((1, ML, 1), batch3),
        pl.BlockSpec((1, ML, 1), batch3),
        full_spec(pbins), full_spec(ebins),
        full_spec(pitch_embedding), full_spec(energy_embedding),
    ]
    params_flat = _pack_params(dur_params) + _pack_params(pitch_params) + _pack_params(energy_params)
    in_specs += [full_spec(a) for a in params_flat]

    out_shapes = [
        jax.ShapeDtypeStruct((B, ML, D), jnp.float32),
        jax.ShapeDtypeStruct((B, S, 1), jnp.float32),
        jax.ShapeDtypeStruct((B, ML, 1), jnp.float32),
        jax.ShapeDtypeStruct((B, ML, 1), jnp.float32),
        jax.ShapeDtypeStruct((B, 1, 128), jnp.float32),
    ]
    out_specs = [
        pl.BlockSpec((1, ML, D), batch3),
        pl.BlockSpec((1, S, 1), batch3),
        pl.BlockSpec((1, ML, 1), batch3),
        pl.BlockSpec((1, ML, 1), batch3),
        pl.BlockSpec((1, 1, 128), batch3),
    ]

    x_out, logdur3, ppred3, epred3, mel3 = pl.pallas_call(
        _body,
        grid=(B,),
        in_specs=in_specs,
        out_specs=out_specs,
        out_shape=out_shapes,
    )(x, x_features, dur3, pt3, et3, pbins, ebins,
      pitch_embedding, energy_embedding, *params_flat)

    log_duration_prediction = logdur3.reshape(B, S)
    pitch_prediction = ppred3.reshape(B, ML)
    energy_prediction = epred3.reshape(B, ML)
    mel_len = mel3[:, 0, 127].astype(jnp.int32)

    return (x_out, log_duration_prediction, duration_target, pitch_prediction,
            energy_prediction, mel_len, mel_mask)
